# baseline (speedup 1.0000x reference)
import jax
import jax.numpy as jnp
from jax.experimental import pallas as pl
from jax.experimental.pallas import tpu as pltpu

THRESH = 0.5
ALPHA = 0.1

_NBLK = 1024


def _accum(xv, yv, m2):
    u = jnp.where(m2, yv - xv, 0.0)
    v = jnp.where(m2, xv, 0.0)
    p1 = jnp.sum(u * u, axis=0)
    p2 = jnp.sum(v * v, axis=0)
    p3 = jnp.sum(xv * xv, axis=0)
    return p1, p2, p3


def _body(x_ref, y_ref, o_ref, *, n_total):
    j = pl.program_id(1)
    nj = pl.num_programs(1)

    @pl.when(jnp.logical_and(pl.program_id(0) == 0, j == 0))
    def _():
        o_ref[...] = jnp.zeros_like(o_ref)

    N, C = _NBLK, x_ref.shape[2]
    xv = x_ref[0].reshape(8, 128, C)
    yv = y_ref[0].reshape(8, 128, C)
    m = yv[:, :, 0:1] > THRESH

    @pl.when(j < nj - 1)
    def _():
        mf = m.astype(jnp.float32)
        t = yv - xv
        u = t * mf
        v = xv * mf
        o_ref[0] += jnp.sum(u * t, axis=0)
        o_ref[1] += jnp.sum(v * xv, axis=0)
        o_ref[2] += jnp.sum(xv * xv, axis=0)

    @pl.when(j == nj - 1)
    def _():
        valid = n_total - j * N
        rid = (jax.lax.broadcasted_iota(jnp.int32, (8, 128, 1), 0) * 128
               + jax.lax.broadcasted_iota(jnp.int32, (8, 128, 1), 1))
        rowv = rid < valid
        p1, p2, p3 = _accum(jnp.where(rowv, xv, 0.0), yv,
                            jnp.logical_and(m, rowv))
        o_ref[0] += p1
        o_ref[1] += p2
        o_ref[2] += p3


def kernel(x, y):
    import functools
    B, N, C = x.shape
    nj = (N + _NBLK - 1) // _NBLK
    out = pl.pallas_call(
        functools.partial(_body, n_total=N),
        grid=(B, nj),
        in_specs=[
            pl.BlockSpec((1, _NBLK, C), lambda i, j: (i, j, 0)),
            pl.BlockSpec((1, _NBLK, C), lambda i, j: (i, j, 0)),
        ],
        out_specs=pl.BlockSpec((3, 128, C), lambda i, j: (0, 0, 0)),
        out_shape=jax.ShapeDtypeStruct((3, 128, C), jnp.float32),
    )(x, y)
    return (jnp.sum(out[0]) - ALPHA * jnp.sum(out[1])
            + ALPHA * jnp.sum(out[2][:, 0]))
